# baseline (speedup 1.0000x reference)
import functools

import jax
import jax.numpy as jnp
from jax import lax
from jax.experimental import pallas as pl

_TILE = 2048
_NEG_INF = float("-inf")


def _route(logits, n_experts):
    e = n_experts
    tri = (lax.broadcasted_iota(jnp.int32, (e, e), 0)
           < lax.broadcasted_iota(jnp.int32, (e, e), 1)).astype(jnp.float32)
    col = lax.broadcasted_iota(jnp.int32, (e, 1), 0).astype(jnp.float32)

    def first_argmax(vals, m):
        eq = (vals == m).astype(jnp.float32)
        prefix = jnp.dot(eq, tri, preferred_element_type=jnp.float32)
        onehot = eq * (prefix == 0.0).astype(jnp.float32)
        idx = jnp.dot(onehot, col, preferred_element_type=jnp.float32)
        return onehot, idx

    m1 = jnp.max(logits, axis=-1, keepdims=True)
    oh1, i1 = first_argmax(logits, m1)
    l2 = jnp.where(oh1 > 0.0, _NEG_INF, logits)
    m2 = jnp.max(l2, axis=-1, keepdims=True)
    _, i2 = first_argmax(l2, m2)
    r = jnp.exp(m2 - m1)
    w1 = 1.0 / (1.0 + r)
    idx = jnp.concatenate([i1, i2], axis=-1).astype(jnp.int32)
    w = jnp.concatenate([w1, r * w1], axis=-1)
    return idx, w


def _router_body(xa_ref, xb_ref, wt_ref, ia_ref, ib_ref, wa_ref, wb_ref,
                 *, n_experts):
    wt = wt_ref[...]
    la = jnp.dot(xa_ref[0], wt, preferred_element_type=jnp.float32)
    lb = jnp.dot(xb_ref[0], wt, preferred_element_type=jnp.float32)
    ia, wa = _route(la, n_experts)
    ib, wb = _route(lb, n_experts)
    ia_ref[0] = ia
    ib_ref[0] = ib
    wa_ref[0] = wa
    wb_ref[0] = wb


def kernel(x, weight):
    n_experts, dim = weight.shape
    xf = x.reshape(-1, dim)
    n = xf.shape[0]
    nblk = n // _TILE
    grid = nblk // 2
    x3 = xf.reshape(nblk, _TILE, dim)
    body = functools.partial(_router_body, n_experts=n_experts)
    half = jax.ShapeDtypeStruct((grid, _TILE, 2), jnp.int32)
    halff = jax.ShapeDtypeStruct((grid, _TILE, 2), jnp.float32)
    ia, ib, wa, wb = pl.pallas_call(
        body,
        grid=(grid,),
        in_specs=[
            pl.BlockSpec((1, _TILE, dim), lambda i: (2 * i, 0, 0)),
            pl.BlockSpec((1, _TILE, dim), lambda i: (2 * i + 1, 0, 0)),
            pl.BlockSpec((dim, n_experts), lambda i: (0, 0)),
        ],
        out_specs=[
            pl.BlockSpec((1, _TILE, 2), lambda i: (i, 0, 0)),
            pl.BlockSpec((1, _TILE, 2), lambda i: (i, 0, 0)),
            pl.BlockSpec((1, _TILE, 2), lambda i: (i, 0, 0)),
            pl.BlockSpec((1, _TILE, 2), lambda i: (i, 0, 0)),
        ],
        out_shape=[half, half, halff, halff],
    )(x3, x3, weight.T)
    idx = jnp.stack([ia, ib], axis=1).reshape(n, 2)
    w = jnp.stack([wa, wb], axis=1).reshape(n, 2)
    return idx, w


# speedup vs baseline: 1.3178x; 1.3178x over previous
import functools

import jax
import jax.numpy as jnp
from jax import lax
from jax.experimental import pallas as pl
from jax.experimental.pallas import tpu as pltpu

_TILE = 2048
_NBUF = 4
_NEG_INF = float("-inf")


def _route(logits, n_experts):
    e = n_experts
    tri = (lax.broadcasted_iota(jnp.int32, (e, e), 0)
           < lax.broadcasted_iota(jnp.int32, (e, e), 1)).astype(jnp.float32)
    col = lax.broadcasted_iota(jnp.int32, (e, 1), 0).astype(jnp.float32)

    def first_argmax(vals, m):
        eq = (vals == m).astype(jnp.float32)
        prefix = jnp.dot(eq, tri, preferred_element_type=jnp.float32)
        onehot = eq * (prefix == 0.0).astype(jnp.float32)
        idx = jnp.dot(onehot, col, preferred_element_type=jnp.float32)
        return onehot, idx

    m1 = jnp.max(logits, axis=-1, keepdims=True)
    oh1, i1 = first_argmax(logits, m1)
    l2 = jnp.where(oh1 > 0.0, _NEG_INF, logits)
    m2 = jnp.max(l2, axis=-1, keepdims=True)
    _, i2 = first_argmax(l2, m2)
    r = jnp.exp(m2 - m1)
    w1 = 1.0 / (1.0 + r)
    idx = jnp.concatenate([i1, i2], axis=-1).astype(jnp.int32)
    w = jnp.concatenate([w1, r * w1], axis=-1)
    return idx, w


def _body(x_hbm, wt_ref, idx_hbm, w_hbm, xbufs, ibufs, wbufs,
          in_sems, out_sems, *, n_experts, nsteps):
    def in_copy(step, buf):
        return pltpu.make_async_copy(
            x_hbm.at[pl.ds(step * _TILE, _TILE)], xbufs.at[buf],
            in_sems.at[buf])

    for b in range(_NBUF):
        in_copy(b, b).start()

    def step_fn(i, _):
        buf = lax.rem(i, _NBUF)
        in_copy(i, buf).wait()
        logits = jnp.dot(xbufs[buf], wt_ref[...],
                         preferred_element_type=jnp.float32)
        @pl.when(i + _NBUF < nsteps)
        def _():
            in_copy(i + _NBUF, buf).start()
        idx, w = _route(logits, n_experts)
        @pl.when(i >= _NBUF)
        def _():
            pltpu.make_async_copy(
                ibufs.at[buf], idx_hbm.at[pl.ds((i - _NBUF) * _TILE, _TILE)],
                out_sems.at[buf]).wait()
            pltpu.make_async_copy(
                wbufs.at[buf], w_hbm.at[pl.ds((i - _NBUF) * _TILE, _TILE)],
                out_sems.at[buf]).wait()
        ibufs[buf] = idx
        wbufs[buf] = w
        pltpu.make_async_copy(
            ibufs.at[buf], idx_hbm.at[pl.ds(i * _TILE, _TILE)],
            out_sems.at[buf]).start()
        pltpu.make_async_copy(
            wbufs.at[buf], w_hbm.at[pl.ds(i * _TILE, _TILE)],
            out_sems.at[buf]).start()
        return 0

    lax.fori_loop(0, nsteps, step_fn, 0)
    for b in range(_NBUF):
        i = nsteps - _NBUF + b
        buf = i % _NBUF
        pltpu.make_async_copy(
            ibufs.at[buf], idx_hbm.at[pl.ds(i * _TILE, _TILE)],
            out_sems.at[buf]).wait()
        pltpu.make_async_copy(
            wbufs.at[buf], w_hbm.at[pl.ds(i * _TILE, _TILE)],
            out_sems.at[buf]).wait()


def kernel(x, weight):
    n_experts, dim = weight.shape
    xf = x.reshape(-1, dim)
    n = xf.shape[0]
    nsteps = n // _TILE
    body = functools.partial(_body, n_experts=n_experts, nsteps=nsteps)
    idx, w = pl.pallas_call(
        body,
        in_specs=[
            pl.BlockSpec(memory_space=pl.ANY),
            pl.BlockSpec(memory_space=pltpu.VMEM),
        ],
        out_specs=[
            pl.BlockSpec(memory_space=pl.ANY),
            pl.BlockSpec(memory_space=pl.ANY),
        ],
        out_shape=[
            jax.ShapeDtypeStruct((n, 2), jnp.int32),
            jax.ShapeDtypeStruct((n, 2), jnp.float32),
        ],
        scratch_shapes=[
            pltpu.VMEM((_NBUF, _TILE, dim), jnp.float32),
            pltpu.VMEM((_NBUF, _TILE, 2), jnp.int32),
            pltpu.VMEM((_NBUF, _TILE, 2), jnp.float32),
            pltpu.SemaphoreType.DMA((_NBUF,)),
            pltpu.SemaphoreType.DMA((_NBUF,)),
        ],
    )(xf, weight.T)
    return idx, w
